# baseline (speedup 1.0000x reference)
import functools

import jax
import jax.numpy as jnp
from jax import lax
from jax.experimental import pallas as pl
from jax.experimental.pallas import tpu as pltpu
from jax.experimental.pallas import tpu_sc as plsc

_C = 128
_P = 8
_D = 256
_STRIDE = 8
_COEF = 0.999
_LOSS_W = 0.1
_EPS = 1e-12

_SEG_BW = 16384

_NC = 2
_NS = 16
_L = 16
_NW = _NC * _NS
_NPROTO = _C * _P
_PP = _NPROTO // _NW
_ROWS = 64
_DG = _D // _L


def _bank_sc_body(mb_hbm, out_hbm, buf0, buf1, stage, sem0, sem1):
    wid = lax.axis_index("s") * _NC + lax.axis_index("c")
    base = wid * _PP
    bufs = (buf0, buf1)
    sems = (sem0, sem1)
    pltpu.async_copy(mb_hbm.at[base], buf0, sem0)

    @pl.loop(0, _PP, step=2)
    def _outer(k):
        for par in range(2):
            p = k + par
            nxt = p + 1

            @pl.when(nxt < _PP)
            def _():
                pltpu.async_copy(mb_hbm.at[base + nxt], bufs[1 - par],
                                 sems[1 - par])

            buf = bufs[par]
            pltpu.make_async_copy(mb_hbm.at[base + p], buf, sems[par]).wait()

            def row_body(r, carry):
                return tuple(carry[g] + buf[r, pl.ds(g * _L, _L)]
                             for g in range(_DG))

            init = tuple(jnp.zeros((_L,), jnp.float32) for _ in range(_DG))
            sums = lax.fori_loop(0, _ROWS, row_body, init)
            for g in range(_DG):
                stage[p, pl.ds(g * _L, _L)] = sums[g]

    pltpu.sync_copy(stage, out_hbm.at[pl.ds(base, _PP)])


@functools.partial(
    pl.kernel,
    out_type=jax.ShapeDtypeStruct((_NPROTO, _D), jnp.float32),
    mesh=plsc.VectorSubcoreMesh(core_axis_name="c", subcore_axis_name="s"),
    scratch_types=[
        pltpu.VMEM((_ROWS, _D), jnp.float32),
        pltpu.VMEM((_ROWS, _D), jnp.float32),
        pltpu.VMEM((_PP, _D), jnp.float32),
        pltpu.SemaphoreType.DMA,
        pltpu.SemaphoreType.DMA,
    ],
)
def _bank_sums_sc(mb_hbm, out_hbm, buf0, buf1, stage, sem0, sem1):
    _bank_sc_body(mb_hbm, out_hbm, buf0, buf1, stage, sem0, sem1)


def _seg_ce_kernel(x_ref, t_ref, acc_ref):
    i = pl.program_id(0)
    j = pl.program_id(1)
    x = x_ref[0]
    t = t_ref[0, 0, 0]
    m = jnp.max(x, axis=0)
    s = jnp.sum(jnp.exp(x - m[None, :]), axis=0)
    lse = m + jnp.log(s)
    cls = jax.lax.broadcasted_iota(jnp.int32, x.shape, 0)
    tsel = jnp.sum(jnp.where(cls == t[None, :], x, 0.0), axis=0)
    partial = jnp.sum(lse - tsel)

    @pl.when(jnp.logical_and(i == 0, j == 0))
    def _():
        acc_ref[0, 0] = 0.0

    acc_ref[0, 0] += partial


def _blend_kernel(sums_ref, proto_ref, out_ref):
    mean = sums_ref[...] * (1.0 / _ROWS)
    n1 = jnp.sqrt(jnp.sum(mean * mean, axis=-1, keepdims=True))
    nm = mean / jnp.maximum(n1, _EPS)
    blended = nm * (1.0 - _COEF) + proto_ref[...] * _COEF
    n2 = jnp.sqrt(jnp.sum(blended * blended, axis=-1, keepdims=True))
    out_ref[...] = blended / jnp.maximum(n2, _EPS)


def _contrast_kernel(emb_ref, protos_ref, lb_ref, acc_ref):
    i = pl.program_id(0)
    logits = jax.lax.dot_general(
        emb_ref[...], protos_ref[...],
        dimension_numbers=(((1,), (1,)), ((), ())),
        preferred_element_type=jnp.float32)
    lb = lb_ref[0, 0]
    m = jnp.max(logits, axis=1)
    s = jnp.sum(jnp.exp(logits - m[:, None]), axis=1)
    lse = m + jnp.log(s)
    col = jax.lax.broadcasted_iota(jnp.int32, logits.shape, 1)
    lo = (lb * _P)[:, None]
    mask = jnp.logical_and(col >= lo, col < lo + _P)
    clsmax = jnp.max(jnp.where(mask, logits, -jnp.inf), axis=1)
    partial = jnp.sum(lse - clsmax)

    @pl.when(i == 0)
    def _():
        acc_ref[0, 0] = 0.0

    acc_ref[0, 0] += partial


def kernel(logits, embedding, memory_bank, prototypes, target, dataset_ids):
    b, c, h, w = logits.shape
    hw = h * w
    n_seg = b * hw
    nb = hw // _SEG_BW

    bank_sums = _bank_sums_sc(memory_bank)

    logits_r = logits.reshape(b, c, hw)
    target_r = target.reshape(b, nb, 1, _SEG_BW)

    seg_sum = pl.pallas_call(
        _seg_ce_kernel,
        grid=(b, nb),
        in_specs=[
            pl.BlockSpec((1, c, _SEG_BW), lambda i, j: (i, 0, j)),
            pl.BlockSpec((1, 1, 1, _SEG_BW), lambda i, j: (i, j, 0, 0)),
        ],
        out_specs=pl.BlockSpec(memory_space=pltpu.SMEM),
        out_shape=jax.ShapeDtypeStruct((1, 1), jnp.float32),
    )(logits_r, target_r)

    protos = pl.pallas_call(
        _blend_kernel,
        in_specs=[
            pl.BlockSpec((_NPROTO, _D), lambda: (0, 0)),
            pl.BlockSpec((_NPROTO, _D), lambda: (0, 0)),
        ],
        out_specs=pl.BlockSpec((_NPROTO, _D), lambda: (0, 0)),
        out_shape=jax.ShapeDtypeStruct((_NPROTO, _D), jnp.float32),
    )(bank_sums, prototypes)

    rearr_emb = jnp.transpose(embedding, (0, 2, 3, 1)).reshape(-1, _D)
    n_ctr = rearr_emb.shape[0]
    contrast_lb = target[:, ::_STRIDE, ::_STRIDE].reshape(-1)
    _R = 512
    nr = n_ctr // _R
    lb_r = contrast_lb.reshape(nr, 1, _R)

    ctr_sum = pl.pallas_call(
        _contrast_kernel,
        grid=(nr,),
        in_specs=[
            pl.BlockSpec((_R, _D), lambda i: (i, 0)),
            pl.BlockSpec((_NPROTO, _D), lambda i: (0, 0)),
            pl.BlockSpec((1, 1, _R), lambda i: (i, 0, 0)),
        ],
        out_specs=pl.BlockSpec(memory_space=pltpu.SMEM),
        out_shape=jax.ShapeDtypeStruct((1, 1), jnp.float32),
    )(rearr_emb, protos, lb_r)

    loss_seg = seg_sum[0, 0] / n_seg
    loss_contrast = ctr_sum[0, 0] / n_ctr
    loss = loss_seg + _LOSS_W * loss_contrast
    return (loss, loss_seg, loss_contrast, protos)


# speedup vs baseline: 1.2428x; 1.2428x over previous
import functools

import jax
import jax.numpy as jnp
from jax import lax
from jax.experimental import pallas as pl
from jax.experimental.pallas import tpu as pltpu
from jax.experimental.pallas import tpu_sc as plsc

_C = 128
_P = 8
_D = 256
_STRIDE = 8
_COEF = 0.999
_LOSS_W = 0.1
_EPS = 1e-12

_SEG_CB = 16
_HW0 = 8
_HW1 = 18432

_NC = 2
_NS = 16
_L = 16
_NW = _NC * _NS
_NPROTO = _C * _P
_PP = _NPROTO // _NW
_ROWS = 64
_DG = _D // _L


def _bank_sc_body(mb_hbm, out_hbm, buf0, buf1, stage, sem0, sem1):
    wid = lax.axis_index("s") * _NC + lax.axis_index("c")
    base = wid * _PP
    bufs = (buf0, buf1)
    sems = (sem0, sem1)
    pltpu.async_copy(mb_hbm.at[base], buf0, sem0)

    @pl.loop(0, _PP, step=2)
    def _outer(k):
        for par in range(2):
            p = k + par
            nxt = p + 1

            @pl.when(nxt < _PP)
            def _():
                pltpu.async_copy(mb_hbm.at[base + nxt], bufs[1 - par],
                                 sems[1 - par])

            buf = bufs[par]
            pltpu.make_async_copy(mb_hbm.at[base + p], buf, sems[par]).wait()

            def row_body(r, carry):
                return tuple(carry[g] + buf[r, pl.ds(g * _L, _L)]
                             for g in range(_DG))

            init = tuple(jnp.zeros((_L,), jnp.float32) for _ in range(_DG))
            sums = lax.fori_loop(0, _ROWS, row_body, init)
            for g in range(_DG):
                stage[p, pl.ds(g * _L, _L)] = sums[g]

    pltpu.sync_copy(stage, out_hbm.at[pl.ds(base, _PP)])


@functools.partial(
    pl.kernel,
    out_type=jax.ShapeDtypeStruct((_NPROTO, _D), jnp.float32),
    mesh=plsc.VectorSubcoreMesh(core_axis_name="c", subcore_axis_name="s"),
    scratch_types=[
        pltpu.VMEM((_ROWS, _D), jnp.float32),
        pltpu.VMEM((_ROWS, _D), jnp.float32),
        pltpu.VMEM((_PP, _D), jnp.float32),
        pltpu.SemaphoreType.DMA,
        pltpu.SemaphoreType.DMA,
    ],
)
def _bank_sums_sc(mb_hbm, out_hbm, buf0, buf1, stage, sem0, sem1):
    _bank_sc_body(mb_hbm, out_hbm, buf0, buf1, stage, sem0, sem1)


def _seg_ce_kernel(x_ref, t_ref, acc_ref, m_acc, s_acc, g_acc):
    i = pl.program_id(0)
    cb = pl.program_id(1)
    ncb = pl.num_programs(1)
    x = x_ref[0]
    t = t_ref[0]

    @pl.when(cb == 0)
    def _():
        m_acc[...] = jnp.full((_HW0, _HW1), -jnp.inf, jnp.float32)
        s_acc[...] = jnp.zeros((_HW0, _HW1), jnp.float32)
        g_acc[...] = jnp.zeros((_HW0, _HW1), jnp.float32)

    cls = cb * _SEG_CB + jax.lax.broadcasted_iota(jnp.int32, x.shape, 0)
    m_old = m_acc[...]
    m_new = jnp.maximum(m_old, jnp.max(x, axis=0))
    s_acc[...] = (s_acc[...] * jnp.exp(m_old - m_new)
                  + jnp.sum(jnp.exp(x - m_new[None]), axis=0))
    m_acc[...] = m_new
    g_acc[...] += jnp.sum(jnp.where(cls == t[None], x, 0.0), axis=0)

    @pl.when(jnp.logical_and(i == 0, cb == 0))
    def _():
        acc_ref[0, 0] = 0.0

    @pl.when(cb == ncb - 1)
    def _():
        acc_ref[0, 0] += jnp.sum(m_acc[...] + jnp.log(s_acc[...]) - g_acc[...])


def _blend_kernel(sums_ref, proto_ref, out_ref):
    mean = sums_ref[...] * (1.0 / _ROWS)
    n1 = jnp.sqrt(jnp.sum(mean * mean, axis=-1, keepdims=True))
    nm = mean / jnp.maximum(n1, _EPS)
    blended = nm * (1.0 - _COEF) + proto_ref[...] * _COEF
    n2 = jnp.sqrt(jnp.sum(blended * blended, axis=-1, keepdims=True))
    out_ref[...] = blended / jnp.maximum(n2, _EPS)


def _contrast_kernel(emb_ref, protos_ref, lb_ref, acc_ref):
    i = pl.program_id(0)
    logits = jax.lax.dot_general(
        emb_ref[...], protos_ref[...],
        dimension_numbers=(((1,), (1,)), ((), ())),
        preferred_element_type=jnp.float32)
    lb = lb_ref[0, 0]
    m = jnp.max(logits, axis=1)
    s = jnp.sum(jnp.exp(logits - m[:, None]), axis=1)
    lse = m + jnp.log(s)
    col = jax.lax.broadcasted_iota(jnp.int32, logits.shape, 1)
    lo = (lb * _P)[:, None]
    mask = jnp.logical_and(col >= lo, col < lo + _P)
    clsmax = jnp.max(jnp.where(mask, logits, -jnp.inf), axis=1)
    partial = jnp.sum(lse - clsmax)

    @pl.when(i == 0)
    def _():
        acc_ref[0, 0] = 0.0

    acc_ref[0, 0] += partial


def kernel(logits, embedding, memory_bank, prototypes, target, dataset_ids):
    b, c, h, w = logits.shape
    hw = h * w
    n_seg = b * hw

    bank_sums = _bank_sums_sc(memory_bank)

    logits_r = logits.reshape(b, c, _HW0, _HW1)
    target_r = target.reshape(b, _HW0, _HW1)

    seg_sum = pl.pallas_call(
        _seg_ce_kernel,
        grid=(b, c // _SEG_CB),
        in_specs=[
            pl.BlockSpec((1, _SEG_CB, _HW0, _HW1), lambda i, cb: (i, cb, 0, 0)),
            pl.BlockSpec((1, _HW0, _HW1), lambda i, cb: (i, 0, 0)),
        ],
        out_specs=pl.BlockSpec(memory_space=pltpu.SMEM),
        out_shape=jax.ShapeDtypeStruct((1, 1), jnp.float32),
        scratch_shapes=[
            pltpu.VMEM((_HW0, _HW1), jnp.float32),
            pltpu.VMEM((_HW0, _HW1), jnp.float32),
            pltpu.VMEM((_HW0, _HW1), jnp.float32),
        ],
    )(logits_r, target_r)

    protos = pl.pallas_call(
        _blend_kernel,
        in_specs=[
            pl.BlockSpec((_NPROTO, _D), lambda: (0, 0)),
            pl.BlockSpec((_NPROTO, _D), lambda: (0, 0)),
        ],
        out_specs=pl.BlockSpec((_NPROTO, _D), lambda: (0, 0)),
        out_shape=jax.ShapeDtypeStruct((_NPROTO, _D), jnp.float32),
    )(bank_sums, prototypes)

    rearr_emb = jnp.transpose(embedding, (0, 2, 3, 1)).reshape(-1, _D)
    n_ctr = rearr_emb.shape[0]
    contrast_lb = target[:, ::_STRIDE, ::_STRIDE].reshape(-1)
    _R = 512
    nr = n_ctr // _R
    lb_r = contrast_lb.reshape(nr, 1, _R)

    ctr_sum = pl.pallas_call(
        _contrast_kernel,
        grid=(nr,),
        in_specs=[
            pl.BlockSpec((_R, _D), lambda i: (i, 0)),
            pl.BlockSpec((_NPROTO, _D), lambda i: (0, 0)),
            pl.BlockSpec((1, 1, _R), lambda i: (i, 0, 0)),
        ],
        out_specs=pl.BlockSpec(memory_space=pltpu.SMEM),
        out_shape=jax.ShapeDtypeStruct((1, 1), jnp.float32),
    )(rearr_emb, protos, lb_r)

    loss_seg = seg_sum[0, 0] / n_seg
    loss_contrast = ctr_sum[0, 0] / n_ctr
    loss = loss_seg + _LOSS_W * loss_contrast
    return (loss, loss_seg, loss_contrast, protos)
